# baseline (speedup 1.0000x reference)
import jax
import jax.numpy as jnp
from jax import lax
from jax.experimental import pallas as pl
from jax.experimental.pallas import tpu as pltpu
from jax.experimental.pallas import tpu_sc as plsc

B = 4
S = 2048
N_BN = 32
D = 1024
SEQ_OUT = S + N_BN
NW = 32
IDX_PER_W = (B * S) // NW
CW = 16
NCHUNK = IDX_PER_W // CW
NB = 6
S_PER_W = S // (NW // B)


def kernel(lang_x, embedding_table, bn_embedding):
    idx = lang_x.reshape(B * S)
    mesh = plsc.VectorSubcoreMesh(core_axis_name="c", subcore_axis_name="s")

    @pl.kernel(
        out_type=jax.ShapeDtypeStruct((B * SEQ_OUT, D), embedding_table.dtype),
        mesh=mesh,
        scratch_types=[
            pltpu.VMEM((IDX_PER_W,), jnp.int32),
            pltpu.VMEM((CW, D), jnp.float32),
            pltpu.VMEM((CW, D), jnp.float32),
            pltpu.VMEM((CW, D), jnp.float32),
            pltpu.VMEM((CW, D), jnp.float32),
            pltpu.VMEM((CW, D), jnp.float32),
            pltpu.VMEM((CW, D), jnp.float32),
            pltpu.VMEM((8, D), jnp.float32),
            pltpu.SemaphoreType.DMA,
            pltpu.SemaphoreType.DMA,
            pltpu.SemaphoreType.DMA,
            pltpu.SemaphoreType.DMA,
            pltpu.SemaphoreType.DMA,
            pltpu.SemaphoreType.DMA,
            pltpu.SemaphoreType.DMA,
            pltpu.SemaphoreType.DMA,
            pltpu.SemaphoreType.DMA,
            pltpu.SemaphoreType.DMA,
            pltpu.SemaphoreType.DMA,
            pltpu.SemaphoreType.DMA,
            pltpu.SemaphoreType.DMA,
        ],
    )
    def emb_kernel(
        table_hbm, idx_hbm, bn_hbm, out_hbm,
        idx_v, rows_a, rows_b, rows_c, rows_d, rows_e, rows_f, bn_v,
        sem_ga, sem_gb, sem_gc, sem_gd, sem_ge, sem_gf,
        sem_oa, sem_ob, sem_oc, sem_od, sem_oe, sem_of, sem_bn,
    ):
        wid = lax.axis_index("s") * 2 + lax.axis_index("c")
        base = wid * IDX_PER_W
        batch = wid // (NW // B)
        row0 = batch * SEQ_OUT + (wid % (NW // B)) * S_PER_W

        pltpu.sync_copy(idx_hbm.at[pl.ds(base, IDX_PER_W)], idx_v)

        bufs = (rows_a, rows_b, rows_c, rows_d, rows_e, rows_f)
        gsems = (sem_ga, sem_gb, sem_gc, sem_gd, sem_ge, sem_gf)
        osems = (sem_oa, sem_ob, sem_oc, sem_od, sem_oe, sem_of)

        def gath(c):
            return pltpu.async_copy(
                table_hbm.at[idx_v.at[pl.ds(c * CW, CW)]],
                bufs[c % NB],
                gsems[c % NB],
            )

        gathers = [None] * NCHUNK
        outs = [None] * NCHUNK
        for _c in range(NB - 1):
            gathers[_c] = gath(_c)

        @pl.when(wid < 16)
        def _():
            b = wid // 4
            j = wid % 4
            pltpu.async_copy(bn_hbm.at[pl.ds(j * 8, 8)], bn_v, sem_bn).wait()
            pltpu.async_copy(
                bn_v, out_hbm.at[pl.ds(b * SEQ_OUT + S + j * 8, 8)], sem_bn
            ).wait()

        LA = NB - 1
        for c in range(NCHUNK):
            if c + LA < NCHUNK:
                if c >= 1:
                    outs[c - 1].wait()
                gathers[c + LA] = gath(c + LA)
            gathers[c].wait()
            outs[c] = pltpu.async_copy(
                bufs[c % NB], out_hbm.at[pl.ds(row0 + c * CW, CW)], osems[c % NB]
            )
        for _c in range(max(0, NCHUNK - NB), NCHUNK):
            outs[_c].wait()

    out = emb_kernel(embedding_table, idx, bn_embedding)
    return out.reshape(B, SEQ_OUT, D)
